# baseline (speedup 1.0000x reference)
import functools

import jax
import jax.numpy as jnp
from jax import lax
from jax.experimental import pallas as pl
from jax.experimental.pallas import tpu as pltpu
from jax.experimental.pallas import tpu_sc as plsc

SEQ = 16384
DIM = 128
NC = 2
NS = 16
NW = NC * NS
B_PER_W = SEQ // NW
CHUNK = 128
NCHUNK = B_PER_W // CHUNK


def _sc_gather_partial_sums(x3, emb):
  mesh = plsc.VectorSubcoreMesh(core_axis_name="c", subcore_axis_name="s")

  nv = DIM // 16

  half = B_PER_W // 2

  @functools.partial(
      pl.kernel,
      mesh=mesh,
      out_type=jax.ShapeDtypeStruct((NW, DIM), jnp.float32),
      scratch_types=[
          pltpu.VMEM((B_PER_W,), jnp.int32),
          [pltpu.VMEM((half, DIM), jnp.float32) for _ in range(2)],
          pltpu.VMEM((DIM,), jnp.float32),
          [pltpu.SemaphoreType.DMA for _ in range(2)],
      ],
  )
  def k(x_hbm, emb_hbm, out_hbm, idx_v, bufs, acc_v, sems):
    wid = lax.axis_index("s") * NC + lax.axis_index("c")
    pltpu.sync_copy(x_hbm.at[pl.ds(wid * B_PER_W, B_PER_W)], idx_v)
    cps = [
        pltpu.async_copy(
            emb_hbm.at[idx_v.at[pl.ds(j * half, half)]], bufs[j], sems[j])
        for j in range(2)
    ]

    def accumulate(buf, acc):
      @plsc.parallel_loop(0, half, unroll=2, carry=acc)
      def final(i, c):
        return tuple(c[d] + buf[i, pl.ds(d * 16, 16)] for d in range(nv))

      return final

    acc = tuple(jnp.zeros((16,), jnp.float32) for _ in range(nv))
    for j in range(2):
      cps[j].wait()
      acc = accumulate(bufs[j], acc)
    for d in range(nv):
      acc_v[pl.ds(d * 16, 16)] = acc[d]
    pltpu.sync_copy(acc_v, out_hbm.at[wid])

  return k(x3, emb)


def _tc_mlp(partials, wh, bh2, wo, bo2):
  def body(p_ref, wh_ref, bh_ref, wo_ref, bo_ref, o_ref):
    embed = jnp.sum(p_ref[...], axis=0, keepdims=True) * (1.0 / SEQ)
    h = jax.lax.dot_general(
        embed, wh_ref[...], (((1,), (1,)), ((), ())),
        preferred_element_type=jnp.float32) + bh_ref[...]
    h = jnp.tanh(h)
    o = jax.lax.dot_general(
        h, wo_ref[...], (((1,), (1,)), ((), ())),
        preferred_element_type=jnp.float32) + bo_ref[...]
    m = jnp.max(o, axis=1, keepdims=True)
    e = jnp.exp(o - m)
    o_ref[...] = e / jnp.sum(e, axis=1, keepdims=True)

  return pl.pallas_call(
      body,
      out_shape=jax.ShapeDtypeStruct((1, 20), jnp.float32),
  )(partials, wh, bh2, wo, bo2)


@jax.jit
def kernel(X, emb, Wh, bh, Wo, bo):
  partials = _sc_gather_partial_sums(X.astype(jnp.int32), emb)
  out = _tc_mlp(partials, Wh, bh.reshape(1, -1), Wo, bo.reshape(1, -1))
  return out.reshape(20)


# speedup vs baseline: 1.0195x; 1.0195x over previous
import functools

import jax
import jax.numpy as jnp
from jax import lax
from jax.experimental import pallas as pl
from jax.experimental.pallas import tpu as pltpu
from jax.experimental.pallas import tpu_sc as plsc

SEQ = 16384
DIM = 128
NC = 2
NS = 16
NW = NC * NS
B_PER_W = SEQ // NW
CHUNK = 128
NCHUNK = B_PER_W // CHUNK


def _sc_gather_partial_sums(x3, emb):
  mesh = plsc.VectorSubcoreMesh(core_axis_name="c", subcore_axis_name="s")

  nv = DIM // 16

  @functools.partial(
      pl.kernel,
      mesh=mesh,
      out_type=jax.ShapeDtypeStruct((NW, DIM), jnp.float32),
      scratch_types=[
          pltpu.VMEM((NCHUNK, CHUNK), jnp.int32),
          [pltpu.VMEM((CHUNK, DIM), jnp.float32) for _ in range(2)],
          pltpu.VMEM((DIM,), jnp.float32),
          [pltpu.SemaphoreType.DMA for _ in range(2)],
      ],
  )
  def k(x_hbm, emb_hbm, out_hbm, idx_v, bufs, acc_v, sems):
    wid = lax.axis_index("s") * NC + lax.axis_index("c")
    pltpu.sync_copy(x_hbm.at[wid], idx_v)
    inflight = [
        pltpu.async_copy(emb_hbm.at[idx_v.at[j]], bufs[j % 2], sems[j % 2])
        for j in range(2)
    ]

    def accumulate(buf, acc):
      @plsc.parallel_loop(0, CHUNK, unroll=4, carry=acc)
      def final(i, c):
        return tuple(c[d] + buf[i, pl.ds(d * 16, 16)] for d in range(nv))

      return final

    acc = tuple(jnp.zeros((16,), jnp.float32) for _ in range(nv))
    for j in range(NCHUNK):
      inflight[j % 2].wait()
      acc = accumulate(bufs[j % 2], acc)
      if j + 2 < NCHUNK:
        inflight[j % 2] = pltpu.async_copy(
            emb_hbm.at[idx_v.at[j + 2]], bufs[j % 2], sems[j % 2])
    for d in range(nv):
      acc_v[pl.ds(d * 16, 16)] = acc[d]
    pltpu.sync_copy(acc_v, out_hbm.at[wid])

  return k(x3, emb)


def _tc_mlp(partials, wh, bh2, wo, bo2):
  def body(p_ref, wh_ref, bh_ref, wo_ref, bo_ref, o_ref):
    embed = jnp.sum(p_ref[...], axis=0, keepdims=True) * (1.0 / SEQ)
    h = jax.lax.dot_general(
        embed, wh_ref[...], (((1,), (1,)), ((), ())),
        preferred_element_type=jnp.float32) + bh_ref[...]
    h = jnp.tanh(h)
    o = jax.lax.dot_general(
        h, wo_ref[...], (((1,), (1,)), ((), ())),
        preferred_element_type=jnp.float32) + bo_ref[...]
    m = jnp.max(o, axis=1, keepdims=True)
    e = jnp.exp(o - m)
    o_ref[...] = e / jnp.sum(e, axis=1, keepdims=True)

  return pl.pallas_call(
      body,
      out_shape=jax.ShapeDtypeStruct((1, 20), jnp.float32),
  )(partials, wh, bh2, wo, bo2)


@jax.jit
def kernel(X, emb, Wh, bh, Wo, bo):
  x3 = X.astype(jnp.int32).reshape(NW, NCHUNK, CHUNK)
  partials = _sc_gather_partial_sums(x3, emb)
  out = _tc_mlp(partials, Wh, bh.reshape(1, -1), Wo, bo.reshape(1, -1))
  return out.reshape(20)


# speedup vs baseline: 1.0398x; 1.0199x over previous
import functools

import jax
import jax.numpy as jnp
from jax import lax
from jax.experimental import pallas as pl
from jax.experimental.pallas import tpu as pltpu
from jax.experimental.pallas import tpu_sc as plsc

SEQ = 16384
DIM = 128
NC = 2
NS = 16
NW = NC * NS
B_PER_W = SEQ // NW
CHUNK = 128
NCHUNK = B_PER_W // CHUNK


def _sc_gather_partial_sums(x3, emb):
  mesh = plsc.VectorSubcoreMesh(core_axis_name="c", subcore_axis_name="s")

  nv = DIM // 16
  tail_n = 32

  @functools.partial(
      pl.kernel,
      mesh=mesh,
      out_type=jax.ShapeDtypeStruct((NW, DIM), jnp.float32),
      scratch_types=[
          pltpu.VMEM((NCHUNK, CHUNK), jnp.int32),
          [pltpu.VMEM((CHUNK, DIM), jnp.float32) for _ in range(2)],
          pltpu.VMEM((32, DIM), jnp.float32),
          pltpu.VMEM((DIM,), jnp.float32),
          [pltpu.SemaphoreType.DMA for _ in range(2)],
          pltpu.SemaphoreType.DMA,
          pltpu.SemaphoreType.DMA,
      ],
  )
  def k(x_hbm, emb_hbm, out_hbm, idx_v, bufs, tail_buf, acc_v, sems,
        idx_sem, tail_sem):
    wid = lax.axis_index("s") * NC + lax.axis_index("c")
    idx0 = pltpu.async_copy(x_hbm.at[wid, 0], idx_v.at[0], idx_sem)
    idx_rest = pltpu.async_copy(
        x_hbm.at[wid, pl.ds(1, NCHUNK - 1)],
        idx_v.at[pl.ds(1, NCHUNK - 1)], tail_sem)
    idx0.wait()
    inflight = [pltpu.async_copy(emb_hbm.at[idx_v.at[0]], bufs[0], sems[0])]
    idx_rest.wait()
    inflight.append(
        pltpu.async_copy(emb_hbm.at[idx_v.at[1]], bufs[1], sems[1]))

    def accumulate(buf, n, unroll, acc):
      @plsc.parallel_loop(0, n, unroll=unroll, carry=acc)
      def final(i, c):
        return tuple(c[d] + buf[i, pl.ds(d * 16, 16)] for d in range(nv))

      return final

    acc = tuple(jnp.zeros((16,), jnp.float32) for _ in range(nv))
    tail_cp = None
    for j in range(NCHUNK):
      inflight[j % 2].wait()
      if j + 2 < NCHUNK - 1:
        inflight[j % 2] = pltpu.async_copy(
            emb_hbm.at[idx_v.at[j + 2]], bufs[j % 2], sems[j % 2])
      elif j + 2 == NCHUNK - 1:
        inflight[j % 2] = pltpu.async_copy(
            emb_hbm.at[idx_v.at[j + 2, pl.ds(0, CHUNK - tail_n)]],
            bufs[j % 2].at[pl.ds(0, CHUNK - tail_n)], sems[j % 2])
        tail_cp = pltpu.async_copy(
            emb_hbm.at[idx_v.at[j + 2, pl.ds(CHUNK - tail_n, tail_n)]],
            tail_buf, tail_sem)
      n = CHUNK if j + 1 < NCHUNK else CHUNK - tail_n
      acc = accumulate(bufs[j % 2], n, 4, acc)
    tail_cp.wait()
    acc = accumulate(tail_buf, tail_n, 4, acc)
    for d in range(nv):
      acc_v[pl.ds(d * 16, 16)] = acc[d]
    pltpu.sync_copy(acc_v, out_hbm.at[wid])

  return k(x3, emb)


def _tc_mlp(partials, wh, bh2, wo, bo2):
  def body(p_ref, wh_ref, bh_ref, wo_ref, bo_ref, o_ref):
    embed = jnp.sum(p_ref[...], axis=0, keepdims=True) * (1.0 / SEQ)
    h = jax.lax.dot_general(
        embed, wh_ref[...], (((1,), (1,)), ((), ())),
        preferred_element_type=jnp.float32) + bh_ref[...]
    h = jnp.tanh(h)
    o = jax.lax.dot_general(
        h, wo_ref[...], (((1,), (1,)), ((), ())),
        preferred_element_type=jnp.float32) + bo_ref[...]
    m = jnp.max(o, axis=1, keepdims=True)
    e = jnp.exp(o - m)
    o_ref[...] = e / jnp.sum(e, axis=1, keepdims=True)

  return pl.pallas_call(
      body,
      out_shape=jax.ShapeDtypeStruct((1, 20), jnp.float32),
  )(partials, wh, bh2, wo, bo2)


@jax.jit
def kernel(X, emb, Wh, bh, Wo, bo):
  x3 = X.astype(jnp.int32).reshape(NW, NCHUNK, CHUNK)
  partials = _sc_gather_partial_sums(x3, emb)
  out = _tc_mlp(partials, Wh, bh.reshape(1, -1), Wo, bo.reshape(1, -1))
  return out.reshape(20)


# speedup vs baseline: 1.0471x; 1.0071x over previous
import functools

import jax
import jax.numpy as jnp
from jax import lax
from jax.experimental import pallas as pl
from jax.experimental.pallas import tpu as pltpu
from jax.experimental.pallas import tpu_sc as plsc

SEQ = 16384
DIM = 128
NC = 2
NS = 16
NW = NC * NS

N_A = 464
N_B = 560
CHUNKS_A = (112, 112, 112, 96)
TAIL_A = 32
CHUNKS_B = (128, 128, 128, 128)
TAIL_B = 48


def _sc_gather_partial_sums(x, emb):
  mesh = plsc.VectorSubcoreMesh(core_axis_name="c", subcore_axis_name="s")

  nv = DIM // 16

  @functools.partial(
      pl.kernel,
      mesh=mesh,
      out_type=jax.ShapeDtypeStruct((NW, DIM), jnp.float32),
      scratch_types=[
          pltpu.VMEM((N_B,), jnp.int32),
          [pltpu.VMEM((CHUNKS_B[0], DIM), jnp.float32) for _ in range(2)],
          pltpu.VMEM((TAIL_B, DIM), jnp.float32),
          pltpu.VMEM((DIM,), jnp.float32),
          [pltpu.SemaphoreType.DMA for _ in range(2)],
          pltpu.SemaphoreType.DMA,
          pltpu.SemaphoreType.DMA,
      ],
  )
  def k(x_hbm, emb_hbm, out_hbm, idx_v, bufs, tail_buf, acc_v, sems,
        idx_sem, tail_sem):
    c = lax.axis_index("c")
    s = lax.axis_index("s")
    wid = s * NC + c

    def accumulate(buf, n, acc):
      @plsc.parallel_loop(0, n, unroll=4, carry=acc)
      def final(i, a):
        return tuple(a[d] + buf[i, pl.ds(d * 16, 16)] for d in range(nv))

      return final

    def emit(chunks, tail_n, base):
      total = sum(chunks) + tail_n
      offs = [0]
      for n in chunks:
        offs.append(offs[-1] + n)
      cp_i0 = pltpu.async_copy(
          x_hbm.at[pl.ds(base, chunks[0])],
          idx_v.at[pl.ds(0, chunks[0])], idx_sem)
      cp_ir = pltpu.async_copy(
          x_hbm.at[pl.ds(base + chunks[0], total - chunks[0])],
          idx_v.at[pl.ds(chunks[0], total - chunks[0])], tail_sem)
      cp_i0.wait()
      inflight = [
          pltpu.async_copy(
              emb_hbm.at[idx_v.at[pl.ds(0, chunks[0])]],
              bufs[0].at[pl.ds(0, chunks[0])], sems[0])
      ]
      cp_ir.wait()
      inflight.append(
          pltpu.async_copy(
              emb_hbm.at[idx_v.at[pl.ds(offs[1], chunks[1])]],
              bufs[1].at[pl.ds(0, chunks[1])], sems[1]))
      tail_cp = None
      acc = tuple(jnp.zeros((16,), jnp.float32) for _ in range(nv))
      for j, nj in enumerate(chunks):
        inflight[j % 2].wait()
        if j + 2 < len(chunks):
          inflight[j % 2] = pltpu.async_copy(
              emb_hbm.at[idx_v.at[pl.ds(offs[j + 2], chunks[j + 2])]],
              bufs[j % 2].at[pl.ds(0, chunks[j + 2])], sems[j % 2])
        elif j + 2 == len(chunks):
          tail_cp = pltpu.async_copy(
              emb_hbm.at[idx_v.at[pl.ds(offs[-1], tail_n)]],
              tail_buf.at[pl.ds(0, tail_n)], tail_sem)
        acc = accumulate(bufs[j % 2], nj, acc)
      tail_cp.wait()
      acc = accumulate(tail_buf, tail_n, acc)
      for d in range(nv):
        acc_v[pl.ds(d * 16, 16)] = acc[d]
      pltpu.sync_copy(acc_v, out_hbm.at[wid])

    @pl.when(c == 0)
    def _():
      emit(CHUNKS_A, TAIL_A, s * N_A)

    @pl.when(c == 1)
    def _():
      emit(CHUNKS_B, TAIL_B, NS * N_A + s * N_B)

  return k(x, emb)


def _tc_mlp(partials, wh, bh2, wo, bo2):
  def body(p_ref, wh_ref, bh_ref, wo_ref, bo_ref, o_ref):
    embed = jnp.sum(p_ref[...], axis=0, keepdims=True) * (1.0 / SEQ)
    h = jax.lax.dot_general(
        embed, wh_ref[...], (((1,), (1,)), ((), ())),
        preferred_element_type=jnp.float32) + bh_ref[...]
    h = jnp.tanh(h)
    o = jax.lax.dot_general(
        h, wo_ref[...], (((1,), (1,)), ((), ())),
        preferred_element_type=jnp.float32) + bo_ref[...]
    m = jnp.max(o, axis=1, keepdims=True)
    e = jnp.exp(o - m)
    o_ref[...] = e / jnp.sum(e, axis=1, keepdims=True)

  return pl.pallas_call(
      body,
      out_shape=jax.ShapeDtypeStruct((1, 20), jnp.float32),
  )(partials, wh, bh2, wo, bo2)


@jax.jit
def kernel(X, emb, Wh, bh, Wo, bo):
  partials = _sc_gather_partial_sums(X.astype(jnp.int32), emb)
  out = _tc_mlp(partials, Wh, bh.reshape(1, -1), Wo, bo.reshape(1, -1))
  return out.reshape(20)
